# baseline (speedup 1.0000x reference)
import functools

import jax
import jax.numpy as jnp
from jax import lax
from jax.experimental import pallas as pl
from jax.experimental.pallas import tpu as pltpu
from jax.experimental.pallas import tpu_sc as plsc

N = 16777216
NC = 2
NS = 16
L = 16
NW = NC * NS
PER_W = N // NW
CHUNK = 8192
NCHUNK = PER_W // CHUNK
NPAIR = NCHUNK // 2

_mesh = plsc.VectorSubcoreMesh(core_axis_name="c", subcore_axis_name="s")


@functools.partial(
    pl.kernel,
    mesh=_mesh,
    out_type=jax.ShapeDtypeStruct((N,), jnp.float32),
    scratch_types=[
        pltpu.VMEM((CHUNK,), jnp.float32),
        pltpu.VMEM((CHUNK,), jnp.float32),
        pltpu.VMEM((CHUNK,), jnp.float32),
        pltpu.VMEM((CHUNK,), jnp.float32),
        pltpu.VMEM((CHUNK,), jnp.float32),
        pltpu.VMEM((CHUNK,), jnp.float32),
        pltpu.SemaphoreType.DMA,
        pltpu.SemaphoreType.DMA,
        pltpu.SemaphoreType.DMA,
        pltpu.SemaphoreType.DMA,
    ],
)
def _stable_zero_div_sc(x_hbm, y_hbm, out_hbm, xv0, yv0, ov0, xv1, yv1, ov1,
                        gs0, gs1, ss0, ss1):
    wid = lax.axis_index("s") * NC + lax.axis_index("c")
    base = wid * PER_W
    bufs = ((xv0, yv0, ov0, gs0, ss0), (xv1, yv1, ov1, gs1, ss1))

    def start_gathers(ci, b):
        xv, yv, _, gs, _ = bufs[b]
        off = base + ci * CHUNK
        pltpu.async_copy(x_hbm.at[pl.ds(off, CHUNK)], xv, gs)
        pltpu.async_copy(y_hbm.at[pl.ds(off, CHUNK)], yv, gs)

    def wait_gathers(b):
        xv, yv, _, gs, _ = bufs[b]
        pltpu.make_async_copy(x_hbm.at[pl.ds(0, CHUNK)], xv, gs).wait()
        pltpu.make_async_copy(y_hbm.at[pl.ds(0, CHUNK)], yv, gs).wait()

    def wait_scatter(b):
        _, _, ov, _, ss = bufs[b]
        pltpu.make_async_copy(ov, out_hbm.at[pl.ds(0, CHUNK)], ss).wait()

    def compute(b):
        xv, yv, ov, _, _ = bufs[b]

        @plsc.parallel_loop(0, CHUNK, step=L, unroll=4)
        def vec_body(i):
            s = pl.ds(i, L)
            yy = yv[s]
            xx = xv[s]
            ov[s] = jnp.where(yy == 0.0, 0.0, xx / yy)

    def start_scatter(ci, b):
        _, _, ov, _, ss = bufs[b]
        off = base + ci * CHUNK
        pltpu.async_copy(ov, out_hbm.at[pl.ds(off, CHUNK)], ss)

    start_gathers(0, 0)

    def pair_body(pi, carry):
        ci0 = pi * 2
        start_gathers(ci0 + 1, 1)
        wait_gathers(0)

        @pl.when(pi >= 1)
        def _():
            wait_scatter(0)

        compute(0)
        start_scatter(ci0, 0)

        @pl.when(pi < NPAIR - 1)
        def _():
            start_gathers(ci0 + 2, 0)

        wait_gathers(1)

        @pl.when(pi >= 1)
        def _():
            wait_scatter(1)

        compute(1)
        start_scatter(ci0 + 1, 1)
        return carry

    lax.fori_loop(0, NPAIR, pair_body, 0)
    wait_scatter(0)
    wait_scatter(1)


def kernel(x, y):
    return _stable_zero_div_sc(x, y)


# speedup vs baseline: 1.1315x; 1.1315x over previous
import functools

import jax
import jax.numpy as jnp
from jax import lax
from jax.experimental import pallas as pl
from jax.experimental.pallas import tpu as pltpu
from jax.experimental.pallas import tpu_sc as plsc

N = 16777216
NC = 2
NS = 16
L = 16
NW = NC * NS
PER_W = N // NW
CHUNK = 16384
NCHUNK = PER_W // CHUNK
NPAIR = NCHUNK // 2

_mesh = plsc.VectorSubcoreMesh(core_axis_name="c", subcore_axis_name="s")


@functools.partial(
    pl.kernel,
    mesh=_mesh,
    out_type=jax.ShapeDtypeStruct((N,), jnp.float32),
    scratch_types=[
        pltpu.VMEM((CHUNK,), jnp.float32),
        pltpu.VMEM((CHUNK,), jnp.float32),
        pltpu.VMEM((CHUNK,), jnp.float32),
        pltpu.VMEM((CHUNK,), jnp.float32),
        pltpu.VMEM((CHUNK,), jnp.float32),
        pltpu.VMEM((CHUNK,), jnp.float32),
        pltpu.SemaphoreType.DMA,
        pltpu.SemaphoreType.DMA,
        pltpu.SemaphoreType.DMA,
        pltpu.SemaphoreType.DMA,
    ],
)
def _stable_zero_div_sc(x_hbm, y_hbm, out_hbm, xv0, yv0, ov0, xv1, yv1, ov1,
                        gs0, gs1, ss0, ss1):
    wid = lax.axis_index("s") * NC + lax.axis_index("c")
    base = wid * PER_W
    bufs = ((xv0, yv0, ov0, gs0, ss0), (xv1, yv1, ov1, gs1, ss1))

    def start_gathers(ci, b):
        xv, yv, _, gs, _ = bufs[b]
        off = base + ci * CHUNK
        pltpu.async_copy(x_hbm.at[pl.ds(off, CHUNK)], xv, gs)
        pltpu.async_copy(y_hbm.at[pl.ds(off, CHUNK)], yv, gs)

    def wait_gathers(b):
        xv, yv, _, gs, _ = bufs[b]
        pltpu.make_async_copy(x_hbm.at[pl.ds(0, CHUNK)], xv, gs).wait()
        pltpu.make_async_copy(y_hbm.at[pl.ds(0, CHUNK)], yv, gs).wait()

    def wait_scatter(b):
        _, _, ov, _, ss = bufs[b]
        pltpu.make_async_copy(ov, out_hbm.at[pl.ds(0, CHUNK)], ss).wait()

    def compute(b):
        xv, yv, ov, _, _ = bufs[b]

        @plsc.parallel_loop(0, CHUNK, step=L, unroll=8)
        def vec_body(i):
            s = pl.ds(i, L)
            yy = yv[s]
            xx = xv[s]
            ov[s] = jnp.where(yy == 0.0, 0.0, xx / yy)

    def start_scatter(ci, b):
        _, _, ov, _, ss = bufs[b]
        off = base + ci * CHUNK
        pltpu.async_copy(ov, out_hbm.at[pl.ds(off, CHUNK)], ss)

    start_gathers(0, 0)

    def pair_body(pi, carry):
        ci0 = pi * 2
        start_gathers(ci0 + 1, 1)
        wait_gathers(0)

        @pl.when(pi >= 1)
        def _():
            wait_scatter(0)

        compute(0)
        start_scatter(ci0, 0)

        @pl.when(pi < NPAIR - 1)
        def _():
            start_gathers(ci0 + 2, 0)

        wait_gathers(1)

        @pl.when(pi >= 1)
        def _():
            wait_scatter(1)

        compute(1)
        start_scatter(ci0 + 1, 1)
        return carry

    lax.fori_loop(0, NPAIR, pair_body, 0)
    wait_scatter(0)
    wait_scatter(1)


def kernel(x, y):
    return _stable_zero_div_sc(x, y)
